# Initial kernel scaffold; baseline (speedup 1.0000x reference)
import jax
import jax.numpy as jnp
from jax.experimental import pallas as pl


def kernel(anchors, logits, bbox_reg):
    raise NotImplementedError("write your pallas kernel here")


# speedup vs baseline: 14.5175x; 14.5175x over previous
import jax
import jax.numpy as jnp
import numpy as np
from jax.experimental import pallas as pl
from jax.experimental.pallas import tpu as pltpu

_IMG = 1024.0
_PRE_TOP_K = 4000
_POST_NMS_TOP_N = 1000
_NMS_THRESH = 0.7
_LANES = 128


def _decode_all(anchors, deltas):
    TO_REMOVE = 1.0
    wa = anchors[:, 2] - anchors[:, 0] + TO_REMOVE
    ha = anchors[:, 3] - anchors[:, 1] + TO_REMOVE
    cxa = anchors[:, 0] + 0.5 * wa
    cya = anchors[:, 1] + 0.5 * ha
    dx, dy, dw, dh = deltas[:, 0], deltas[:, 1], deltas[:, 2], deltas[:, 3]
    clip = float(np.log(1000.0 / 16.0))
    dw = jnp.minimum(dw, clip)
    dh = jnp.minimum(dh, clip)
    px = dx * wa + cxa
    py = dy * ha + cya
    pw = jnp.exp(dw) * wa
    ph = jnp.exp(dh) * ha
    return jnp.stack(
        [px - 0.5 * pw, py - 0.5 * ph, px + 0.5 * pw - 1.0, py + 0.5 * ph - 1.0],
        axis=-1,
    )


def _nms_body(sc_ref, x1_ref, y1_ref, x2_ref, y2_ref, out_ref, sw_ref):
    rows = sc_ref.shape[0]
    num = rows * _LANES
    scores = sc_ref[...]
    x1 = x1_ref[...]
    y1 = y1_ref[...]
    x2 = x2_ref[...]
    y2 = y2_ref[...]
    fid = (
        jax.lax.broadcasted_iota(jnp.int32, (rows, _LANES), 0) * _LANES
        + jax.lax.broadcasted_iota(jnp.int32, (rows, _LANES), 1)
    ).astype(jnp.float32)

    bits = jax.lax.bitcast_convert_type(scores, jnp.int32)
    K = _PRE_TOP_K

    def bs_val(_, lohi):
        lo, hi = lohi
        mid = (lo + hi) // 2
        cnt = jnp.sum((bits >= mid).astype(jnp.int32))
        ok = cnt >= K
        return jnp.where(ok, mid, lo), jnp.where(ok, hi, mid)

    vstar, _ = jax.lax.fori_loop(
        0, 31, bs_val, (jnp.int32(0), jnp.int32(0x3F800001))
    )
    gt = bits > vstar
    eq = bits == vstar
    need = K - jnp.sum(gt.astype(jnp.int32))

    def bs_idx(_, lohi):
        lo, hi = lohi
        mid = (lo + hi) // 2
        cnt = jnp.sum((eq & (fid < mid.astype(jnp.float32))).astype(jnp.int32))
        ok = cnt >= need
        return jnp.where(ok, lo, mid), jnp.where(ok, mid, hi)

    _, cstar = jax.lax.fori_loop(
        0, 17, bs_idx, (jnp.int32(0), jnp.int32(num))
    )
    selected = gt | (eq & (fid < cstar.astype(jnp.float32)))

    ws = x2 - x1 + 1.0
    hs = y2 - y1 + 1.0
    keep = (ws >= 0.0) & (hs >= 0.0)
    sw_ref[...] = jnp.where(selected & keep, scores, jnp.float32(-1e9))

    area2 = (x2 - x1 + 1.0) * (y2 - y1 + 1.0)
    lane = jax.lax.broadcasted_iota(jnp.int32, (1, _LANES), 1).astype(jnp.float32)

    def step(i, carry):
        sw = sw_ref[...]
        m = jnp.max(sw)
        idxf = jnp.min(jnp.where(sw == m, fid, jnp.float32(2e9)))
        onehot = fid == idxf
        bx1 = jnp.sum(jnp.where(onehot, x1, 0.0))
        by1 = jnp.sum(jnp.where(onehot, y1, 0.0))
        bx2 = jnp.sum(jnp.where(onehot, x2, 0.0))
        by2 = jnp.sum(jnp.where(onehot, y2, 0.0))
        bsc = jnp.sum(jnp.where(onehot, scores, 0.0))
        ltx = jnp.maximum(bx1, x1)
        lty = jnp.maximum(by1, y1)
        rbx = jnp.minimum(bx2, x2)
        rby = jnp.minimum(by2, y2)
        ww = jnp.maximum(rbx - ltx + 1.0, 0.0)
        hh = jnp.maximum(rby - lty + 1.0, 0.0)
        inter = ww * hh
        area1 = (bx2 - bx1 + 1.0) * (by2 - by1 + 1.0)
        iou = inter / (area1 + area2 - inter)
        sw_ref[...] = jnp.where((iou > _NMS_THRESH) | onehot, jnp.float32(-1e9), sw)
        valid = m > -1e8
        v0 = jnp.where(valid, bx1, 0.0)
        v1 = jnp.where(valid, by1, 0.0)
        v2 = jnp.where(valid, bx2, 0.0)
        v3 = jnp.where(valid, by2, 0.0)
        v4 = jnp.where(valid, bsc, 0.0)
        row = jnp.where(
            lane == 0.0,
            v0,
            jnp.where(
                lane == 1.0,
                v1,
                jnp.where(
                    lane == 2.0,
                    v2,
                    jnp.where(lane == 3.0, v3, jnp.where(lane == 4.0, v4, 0.0)),
                ),
            ),
        )
        out_ref[pl.ds(i, 1), :] = row
        return carry

    jax.lax.fori_loop(0, _POST_NMS_TOP_N, step, 0)


def _run_one(scores, x1, y1, x2, y2):
    num = scores.shape[0]
    rows = num // _LANES
    out2d = pl.pallas_call(
        _nms_body,
        out_shape=jax.ShapeDtypeStruct((_POST_NMS_TOP_N, _LANES), jnp.float32),
        scratch_shapes=[pltpu.VMEM((rows, _LANES), jnp.float32)],
    )(
        scores.reshape(rows, _LANES),
        x1.reshape(rows, _LANES),
        y1.reshape(rows, _LANES),
        x2.reshape(rows, _LANES),
        y2.reshape(rows, _LANES),
    )
    return out2d[:, :5]


def kernel(anchors, logits, bbox_reg):
    Nb, Ah, Hh, Wh = logits.shape
    lg = jnp.transpose(logits, (0, 2, 3, 1)).reshape(Nb, -1)
    scores = jax.nn.sigmoid(lg)
    reg = (
        bbox_reg.reshape(Nb, Ah, 4, Hh, Wh)
        .transpose(0, 3, 4, 1, 2)
        .reshape(Nb, -1, 4)
    )
    outs = []
    for i in range(Nb):
        boxes = _decode_all(anchors[i], reg[i])
        x1 = jnp.clip(boxes[:, 0], 0.0, _IMG - 1.0)
        y1 = jnp.clip(boxes[:, 1], 0.0, _IMG - 1.0)
        x2 = jnp.clip(boxes[:, 2], 0.0, _IMG - 1.0)
        y2 = jnp.clip(boxes[:, 3], 0.0, _IMG - 1.0)
        outs.append(_run_one(scores[i], x1, y1, x2, y2))
    return jnp.stack(outs, 0)
